# baseline (speedup 1.0000x reference)
import functools

import jax
import jax.numpy as jnp
from jax import lax
from jax.experimental import pallas as pl
from jax.experimental.pallas import tpu as pltpu
import jax.experimental.pallas.tpu_sc as plsc

N = 10000
E = 320000
L = 128
OUT_DIM = 3
MP = 4

ROWS = E // 128
BLK = 2
NW = 32
NPAD = 10240
TSTRIPE = NPAD // 16

NCHUNK = 2
CROWS = ROWS // NCHUNK
CE = CROWS * 128

BE = 1280
BN = 1000

_PREC = lax.Precision.DEFAULT

_mesh = plsc.VectorSubcoreMesh(core_axis_name="c", subcore_axis_name="s")
_sc_params = pltpu.CompilerParams(needs_layout_passes=False)


def _make_gather(nrows):
    nblk = nrows // BLK

    @functools.partial(
        pl.kernel,
        out_type=jax.ShapeDtypeStruct((nrows * 128, L), jnp.float32),
        mesh=_mesh,
        compiler_params=_sc_params,
        scratch_types=[
            pltpu.VMEM((BLK, 128), jnp.int32),
            pltpu.VMEM((BLK, 128), jnp.int32),
            pltpu.VMEM((BLK * 128, L), jnp.float32),
            pltpu.VMEM((BLK * 128, L), jnp.float32),
            pltpu.SemaphoreType.DMA,
        ],
    )
    def gather(ad_hbm, as_hbm, dst_hbm, src_hbm, g_hbm, idx_d, idx_s, rows_d, rows_s, sem):
        wid = lax.axis_index("s") * 2 + lax.axis_index("c")
        nb = (nblk - wid + NW - 1) // NW

        def blk_body(i, carry):
            blk = wid + i * NW
            er = blk * BLK
            eb = er * 128
            pltpu.sync_copy(dst_hbm.at[pl.ds(er, BLK)], idx_d)
            pltpu.sync_copy(src_hbm.at[pl.ds(er, BLK)], idx_s)
            cps = []
            for j in range(BLK):
                cps.append(pltpu.async_copy(ad_hbm.at[idx_d.at[j]], rows_d.at[pl.ds(j * 128, 128)], sem))
                cps.append(pltpu.async_copy(as_hbm.at[idx_s.at[j]], rows_s.at[pl.ds(j * 128, 128)], sem))
            for c in cps:
                c.wait()

            def add_body(r, c2):
                for k in range(L // 16):
                    sl = pl.ds(k * 16, 16)
                    rows_d[r, sl] = rows_d[r, sl] + rows_s[r, sl]
                return c2

            lax.fori_loop(0, BLK * 128, add_body, 0)
            pltpu.sync_copy(rows_d, g_hbm.at[pl.ds(eb, BLK * 128)])
            return carry

        lax.fori_loop(0, nb, blk_body, 0)

    return gather


def _make_scatter(nrows):
    nblk = nrows // BLK

    @functools.partial(
        pl.kernel,
        out_type=(jax.ShapeDtypeStruct((2, NPAD, L), jnp.float32),
                  jax.ShapeDtypeStruct((NW, N), jnp.float32)),
        mesh=_mesh,
        compiler_params=_sc_params,
        scratch_types=[
            pltpu.VMEM((BLK, 128), jnp.int32),
            pltpu.VMEM((BLK, 128), jnp.float32),
            pltpu.VMEM((BLK * 128, L), jnp.float32),
            pltpu.VMEM((N,), jnp.float32),
            pltpu.VMEM_SHARED((NPAD, L), jnp.float32),
        ],
    )
    def scatter(pay_hbm, ex_hbm, dst_hbm, vout_hbm, dout_hbm, idx, exb, rows, den, shared):
        cid = lax.axis_index("c")
        sid = lax.axis_index("s")
        wid = sid * 2 + cid
        base = sid * TSTRIPE

        def zden(i, c):
            den[pl.ds(i * 16, 16)] = jnp.zeros((16,), jnp.float32)
            return c

        lax.fori_loop(0, N // 16, zden, 0)

        def zrows(r, c):
            for k in range(L // 16):
                rows[r, pl.ds(k * 16, 16)] = jnp.zeros((16,), jnp.float32)
            return c

        lax.fori_loop(0, BLK * 128, zrows, 0)
        pltpu.sync_copy(rows.at[pl.ds(0, 256)], shared.at[pl.ds(base, 256)])
        pltpu.sync_copy(rows.at[pl.ds(0, 256)], shared.at[pl.ds(base + 256, 256)])
        pltpu.sync_copy(rows.at[pl.ds(0, 128)], shared.at[pl.ds(base + 512, 128)])
        plsc.subcore_barrier()

        nb = (nblk - wid + NW - 1) // NW

        def blk_body(i, c):
            blk = wid + i * NW
            er = blk * BLK
            eb = er * 128
            pltpu.sync_copy(dst_hbm.at[pl.ds(er, BLK)], idx)
            pltpu.sync_copy(ex_hbm.at[pl.ds(er, BLK)], exb)
            pltpu.sync_copy(pay_hbm.at[pl.ds(eb, BLK * 128)], rows)
            for j in range(BLK):
                for k in range(8):
                    sl = pl.ds(k * 16, 16)
                    plsc.addupdate_scatter(den, [idx[j, sl]], exb[j, sl])
                pltpu.sync_copy(rows.at[pl.ds(j * 128, 128)], shared.at[idx.at[j]], add=True)
            return c

        lax.fori_loop(0, nb, blk_body, 0)
        plsc.subcore_barrier()
        pltpu.sync_copy(shared.at[pl.ds(base, TSTRIPE)], vout_hbm.at[cid, pl.ds(base, TSTRIPE)])
        pltpu.sync_copy(den, dout_hbm.at[wid])

    return scatter


_sc_gather = _make_gather(CROWS)
_sc_scatter = _make_scatter(CROWS)


def _ln_rows(y, g, b):
    mu = jnp.mean(y, axis=-1, keepdims=True)
    var = jnp.mean((y - mu) ** 2, axis=-1, keepdims=True)
    return (y - mu) / jnp.sqrt(var + 1e-5) * g + b


def _full(shape):
    return pl.BlockSpec(shape, lambda i: tuple(0 for _ in shape))


def _tc_encoder(x, W0, b0, Wout, bout, g, b, blk):
    n, din = x.shape

    def body(x_ref, W0_ref, b0_ref, Wout_ref, bout_ref, g_ref, b_ref, out_ref):
        h = jnp.maximum(jnp.dot(x_ref[...], W0_ref[...], precision=_PREC,
                                preferred_element_type=jnp.float32) + b0_ref[...], 0.0)
        y = jnp.dot(h, Wout_ref[...], precision=_PREC,
                    preferred_element_type=jnp.float32) + bout_ref[...]
        out_ref[...] = _ln_rows(y, g_ref[...], b_ref[...])

    return pl.pallas_call(
        body,
        grid=(n // blk,),
        in_specs=[pl.BlockSpec((blk, din), lambda i: (i, 0)),
                  _full((din, L)), _full((1, L)), _full((L, L)), _full((1, L)),
                  _full((1, L)), _full((1, L))],
        out_specs=pl.BlockSpec((blk, L), lambda i: (i, 0)),
        out_shape=jax.ShapeDtypeStruct((n, L), jnp.float32),
    )(x, W0, b0.reshape(1, L), Wout, bout.reshape(1, L), g.reshape(1, L), b.reshape(1, L))


def _tc_node_prod(vlat, Wd, Ws):
    def body(v_ref, wd_ref, ws_ref, ad_ref, as_ref):
        v = v_ref[...]
        ad_ref[...] = jnp.dot(v, wd_ref[...], precision=_PREC, preferred_element_type=jnp.float32)
        as_ref[...] = jnp.dot(v, ws_ref[...], precision=_PREC, preferred_element_type=jnp.float32)

    return pl.pallas_call(
        body,
        grid=(N // BN,),
        in_specs=[pl.BlockSpec((BN, L), lambda i: (i, 0)), _full((L, L)), _full((L, L))],
        out_specs=(pl.BlockSpec((BN, L), lambda i: (i, 0)), pl.BlockSpec((BN, L), lambda i: (i, 0))),
        out_shape=(jax.ShapeDtypeStruct((N, L), jnp.float32),
                   jax.ShapeDtypeStruct((N, L), jnp.float32)),
    )(vlat, Wd, Ws)


def _tc_edge_update(G, elat_num, ex_prev, We, b0, Wout, bout, attW, attb, g, b):
    nE = G.shape[0]

    def body(g_ref, e_ref, exp_ref, We_ref, b0_ref, Wout_ref, bout_ref, aw_ref, ab_ref,
             g_ref2, b_ref2, wne_ref, ex_ref):
        recip = 1.0 / exp_ref[0].T
        el = jnp.concatenate(
            [e_ref[pl.ds(k * 128, 128), :] * recip[:, k:k + 1] for k in range(BE // 128)],
            axis=0)
        hid = jnp.maximum(g_ref[...] + jnp.dot(el, We_ref[...], precision=_PREC,
                                               preferred_element_type=jnp.float32) + b0_ref[...], 0.0)
        eup = jnp.dot(hid, Wout_ref[...], precision=_PREC,
                      preferred_element_type=jnp.float32) + bout_ref[...]
        ne = el + _ln_rows(eup, g_ref2[...], b_ref2[...])
        s = jnp.maximum(jnp.sum(el * aw_ref[...], axis=-1, keepdims=True) + ab_ref[...], 0.0)
        ex = jnp.exp(s)
        wne_ref[...] = ex * ne
        ex_ref[...] = ex.reshape(1, BE // 128, 128)

    return pl.pallas_call(
        body,
        grid=(nE // BE,),
        in_specs=[pl.BlockSpec((BE, L), lambda i: (i, 0)),
                  pl.BlockSpec((BE, L), lambda i: (i, 0)),
                  pl.BlockSpec((1, BE // 128, 128), lambda i: (i, 0, 0)),
                  _full((L, L)), _full((1, L)), _full((L, L)), _full((1, L)),
                  _full((1, L)), _full((1, 1)), _full((1, L)), _full((1, L))],
        out_specs=(pl.BlockSpec((BE, L), lambda i: (i, 0)),
                   pl.BlockSpec((1, BE // 128, 128), lambda i: (i, 0, 0))),
        out_shape=(jax.ShapeDtypeStruct((nE, L), jnp.float32),
                   jax.ShapeDtypeStruct((nE // BE, BE // 128, 128), jnp.float32)),
    )(G, elat_num, ex_prev, We, b0.reshape(1, L), Wout, bout.reshape(1, L),
      attW.reshape(1, L), attb.reshape(1, 1), g.reshape(1, L), b.reshape(1, L))


def _tc_vertex_update(vlat, vps, dps, Wv1, Wv2, b0, Wout, bout, g, b, Wdn, Wsn):
    nv = len(vps)

    def body(v_ref, p0_ref, p1_ref, p2_ref, p3_ref, d0_ref, d1_ref,
             Wv1_ref, Wv2_ref, b0_ref, Wout_ref,
             bout_ref, g_ref, b_ref, wdn_ref, wsn_ref, vo_ref, ad_ref, as_ref):
        v = v_ref[...]
        den = (jnp.sum(d0_ref[...], axis=-1, keepdims=True)
               + jnp.sum(d1_ref[...], axis=-1, keepdims=True))
        vals = p0_ref[...] + p1_ref[...] + p2_ref[...] + p3_ref[...]
        agg = vals / (den + 1e-16)
        hv = jnp.maximum(jnp.dot(v, Wv1_ref[...], precision=_PREC, preferred_element_type=jnp.float32)
                         + jnp.dot(agg, Wv2_ref[...], precision=_PREC, preferred_element_type=jnp.float32)
                         + b0_ref[...], 0.0)
        vup = jnp.dot(hv, Wout_ref[...], precision=_PREC,
                      preferred_element_type=jnp.float32) + bout_ref[...]
        vnew = v + _ln_rows(vup, g_ref[...], b_ref[...])
        vo_ref[...] = vnew
        ad_ref[...] = jnp.dot(vnew, wdn_ref[...], precision=_PREC, preferred_element_type=jnp.float32)
        as_ref[...] = jnp.dot(vnew, wsn_ref[...], precision=_PREC, preferred_element_type=jnp.float32)

    vspec = pl.BlockSpec((BN, L), lambda i: (i, 0))
    dspec = pl.BlockSpec((BN, NW), lambda i: (i, 0))
    return pl.pallas_call(
        body,
        grid=(N // BN,),
        in_specs=[vspec, vspec, vspec, vspec, vspec, dspec, dspec,
                  _full((L, L)), _full((L, L)), _full((1, L)), _full((L, L)),
                  _full((1, L)), _full((1, L)), _full((1, L)), _full((L, L)), _full((L, L))],
        out_specs=(vspec, vspec, vspec),
        out_shape=(jax.ShapeDtypeStruct((N, L), jnp.float32),
                   jax.ShapeDtypeStruct((N, L), jnp.float32),
                   jax.ShapeDtypeStruct((N, L), jnp.float32)),
    )(vlat, *vps, *dps, Wv1, Wv2, b0.reshape(1, L), Wout, bout.reshape(1, L),
      g.reshape(1, L), b.reshape(1, L), Wdn, Wsn)


def _tc_decoder(vlat, W0, b0, Wout, bout):
    def body(v_ref, W0_ref, b0_ref, Wout_ref, bout_ref, out_ref):
        h = jnp.maximum(jnp.dot(v_ref[...], W0_ref[...], precision=_PREC,
                                preferred_element_type=jnp.float32) + b0_ref[...], 0.0)
        out_ref[...] = jnp.dot(h, Wout_ref[...], precision=_PREC,
                               preferred_element_type=jnp.float32) + bout_ref[...]

    Wout_p = jnp.zeros((L, 128), jnp.float32).at[:, :OUT_DIM].set(Wout)
    bout_p = jnp.zeros((1, 128), jnp.float32).at[0, :OUT_DIM].set(bout)
    out = pl.pallas_call(
        body,
        grid=(N // BN,),
        in_specs=[pl.BlockSpec((BN, L), lambda i: (i, 0)),
                  _full((L, L)), _full((1, L)), _full((L, 128)), _full((1, 128))],
        out_specs=pl.BlockSpec((BN, 128), lambda i: (i, 0)),
        out_shape=jax.ShapeDtypeStruct((N, 128), jnp.float32),
    )(vlat, W0, b0.reshape(1, L), Wout_p, bout_p)
    return out[:, :OUT_DIM]


def _fold_bn(p):
    s = p['bn_g'] / jnp.sqrt(p['bn_v'] + 1e-5)
    t = p['bn_b'] - p['bn_m'] * s
    return s[:, None] * p['W0'], t @ p['W0'] + p['b0']


def kernel(x, edge_attr, edge_index, params):
    src_r = edge_index[0].reshape(ROWS, 128)
    dst_r = edge_index[1].reshape(ROWS, 128)
    src_c = [src_r[c * CROWS:(c + 1) * CROWS] for c in range(NCHUNK)]
    dst_c = [dst_r[c * CROWS:(c + 1) * CROWS] for c in range(NCHUNK)]

    W0v, b0v = _fold_bn(params['venc'])
    W0e, b0e = _fold_bn(params['eenc'])
    lng, lnb = params['ln_g'], params['ln_b']

    vlat = _tc_encoder(x, W0v, b0v, params['venc']['Wout'], params['venc']['bout'], lng, lnb, BN)
    elat_c = [
        _tc_encoder(edge_attr[c * CE:(c + 1) * CE], W0e, b0e,
                    params['eenc']['Wout'], params['eenc']['bout'], lng, lnb, BE)
        for c in range(NCHUNK)
    ]

    pp0 = params['proc0']
    Ad, As = _tc_node_prod(vlat, pp0['edge']['W0'][0:L], pp0['edge']['W0'][L:2 * L])

    ex_c = [jnp.ones((CE // BE, BE // 128, 128), jnp.float32) for _ in range(NCHUNK)]

    for i in range(MP):
        pp = params['proc%d' % i]
        ppn = params['proc%d' % ((i + 1) % MP)]
        wne_c, nex_c, vp_list, dp_list = [], [], [], []
        G_c = [None] * NCHUNK
        for c in range(NCHUNK):
            G_c[c] = _sc_gather(Ad, As, dst_c[c], src_c[c])
        for c in range(NCHUNK):
            wne, exr = _tc_edge_update(
                G_c[c], elat_c[c], ex_c[c], pp['edge']['W0'][2 * L:3 * L], pp['edge']['b0'],
                pp['edge']['Wout'], pp['edge']['bout'],
                pp['att_W'][:, 0], pp['att_b'], pp['ln_g'], pp['ln_b'])
            wne_c.append(wne)
            nex_c.append(exr)
            vparts, dparts = _sc_scatter(wne, exr.reshape(CROWS, 128), dst_c[c])
            vp_list.extend([vparts[0, :N], vparts[1, :N]])
            dp_list.append(dparts.T)
        vlat, Ad, As = _tc_vertex_update(
            vlat, vp_list, dp_list,
            pp['vertex']['W0'][0:L], pp['vertex']['W0'][L:2 * L],
            pp['vertex']['b0'], pp['vertex']['Wout'], pp['vertex']['bout'],
            pp['ln_g'], pp['ln_b'],
            ppn['edge']['W0'][0:L], ppn['edge']['W0'][L:2 * L])
        elat_c = wne_c
        ex_c = nex_c

    d = params['dec']
    return _tc_decoder(vlat, d['W0'], d['b0'], d['Wout'], d['bout'])


# speedup vs baseline: 1.0412x; 1.0412x over previous
import functools

import jax
import jax.numpy as jnp
from jax import lax
from jax.experimental import pallas as pl
from jax.experimental.pallas import tpu as pltpu
import jax.experimental.pallas.tpu_sc as plsc

N = 10000
E = 320000
L = 128
OUT_DIM = 3
MP = 4

ROWS = E // 128
BLK = 2
NW = 32
NPAD = 10240
TSTRIPE = NPAD // 16

CHUNK_ROWS = (500, 1250, 750)
NCHUNK = len(CHUNK_ROWS)
CHUNK_OFF = tuple(sum(CHUNK_ROWS[:c]) for c in range(NCHUNK))

BE = 1280
BN = 1000

_PREC = lax.Precision.DEFAULT

_mesh = plsc.VectorSubcoreMesh(core_axis_name="c", subcore_axis_name="s")
_sc_params = pltpu.CompilerParams(needs_layout_passes=False)


def _make_gather(nrows):
    nblk = nrows // BLK

    @functools.partial(
        pl.kernel,
        out_type=jax.ShapeDtypeStruct((nrows * 128, L), jnp.float32),
        mesh=_mesh,
        compiler_params=_sc_params,
        scratch_types=[
            pltpu.VMEM((BLK, 128), jnp.int32),
            pltpu.VMEM((BLK, 128), jnp.int32),
            pltpu.VMEM((BLK * 128, L), jnp.float32),
            pltpu.VMEM((BLK * 128, L), jnp.float32),
            pltpu.SemaphoreType.DMA,
        ],
    )
    def gather(ad_hbm, as_hbm, dst_hbm, src_hbm, g_hbm, idx_d, idx_s, rows_d, rows_s, sem):
        wid = lax.axis_index("s") * 2 + lax.axis_index("c")
        nb = (nblk - wid + NW - 1) // NW

        def blk_body(i, carry):
            blk = wid + i * NW
            er = blk * BLK
            eb = er * 128
            pltpu.sync_copy(dst_hbm.at[pl.ds(er, BLK)], idx_d)
            pltpu.sync_copy(src_hbm.at[pl.ds(er, BLK)], idx_s)
            cps = []
            for j in range(BLK):
                cps.append(pltpu.async_copy(ad_hbm.at[idx_d.at[j]], rows_d.at[pl.ds(j * 128, 128)], sem))
                cps.append(pltpu.async_copy(as_hbm.at[idx_s.at[j]], rows_s.at[pl.ds(j * 128, 128)], sem))
            for c in cps:
                c.wait()

            def add_body(r, c2):
                for k in range(L // 16):
                    sl = pl.ds(k * 16, 16)
                    rows_d[r, sl] = rows_d[r, sl] + rows_s[r, sl]
                return c2

            lax.fori_loop(0, BLK * 128, add_body, 0)
            pltpu.sync_copy(rows_d, g_hbm.at[pl.ds(eb, BLK * 128)])
            return carry

        lax.fori_loop(0, nb, blk_body, 0)

    return gather


def _make_scatter(nrows):
    nblk = nrows // BLK

    @functools.partial(
        pl.kernel,
        out_type=(jax.ShapeDtypeStruct((2, NPAD, L), jnp.float32),
                  jax.ShapeDtypeStruct((NW, N), jnp.float32)),
        mesh=_mesh,
        compiler_params=_sc_params,
        scratch_types=[
            pltpu.VMEM((BLK, 128), jnp.int32),
            pltpu.VMEM((BLK, 128), jnp.float32),
            pltpu.VMEM((BLK * 128, L), jnp.float32),
            pltpu.VMEM((N,), jnp.float32),
            pltpu.VMEM_SHARED((NPAD, L), jnp.float32),
        ],
    )
    def scatter(pay_hbm, ex_hbm, dst_hbm, vout_hbm, dout_hbm, idx, exb, rows, den, shared):
        cid = lax.axis_index("c")
        sid = lax.axis_index("s")
        wid = sid * 2 + cid
        base = sid * TSTRIPE

        def zden(i, c):
            den[pl.ds(i * 16, 16)] = jnp.zeros((16,), jnp.float32)
            return c

        lax.fori_loop(0, N // 16, zden, 0)

        def zrows(r, c):
            for k in range(L // 16):
                rows[r, pl.ds(k * 16, 16)] = jnp.zeros((16,), jnp.float32)
            return c

        lax.fori_loop(0, BLK * 128, zrows, 0)
        pltpu.sync_copy(rows.at[pl.ds(0, 256)], shared.at[pl.ds(base, 256)])
        pltpu.sync_copy(rows.at[pl.ds(0, 256)], shared.at[pl.ds(base + 256, 256)])
        pltpu.sync_copy(rows.at[pl.ds(0, 128)], shared.at[pl.ds(base + 512, 128)])
        plsc.subcore_barrier()

        nb = (nblk - wid + NW - 1) // NW

        def blk_body(i, c):
            blk = wid + i * NW
            er = blk * BLK
            eb = er * 128
            pltpu.sync_copy(dst_hbm.at[pl.ds(er, BLK)], idx)
            pltpu.sync_copy(ex_hbm.at[pl.ds(er, BLK)], exb)
            pltpu.sync_copy(pay_hbm.at[pl.ds(eb, BLK * 128)], rows)
            for j in range(BLK):
                for k in range(8):
                    sl = pl.ds(k * 16, 16)
                    plsc.addupdate_scatter(den, [idx[j, sl]], exb[j, sl])
                pltpu.sync_copy(rows.at[pl.ds(j * 128, 128)], shared.at[idx.at[j]], add=True)
            return c

        lax.fori_loop(0, nb, blk_body, 0)
        plsc.subcore_barrier()
        pltpu.sync_copy(shared.at[pl.ds(base, TSTRIPE)], vout_hbm.at[cid, pl.ds(base, TSTRIPE)])
        pltpu.sync_copy(den, dout_hbm.at[wid])

    return scatter


_sc_gather_c = {r: _make_gather(r) for r in set(CHUNK_ROWS)}
_sc_scatter_c = {r: _make_scatter(r) for r in set(CHUNK_ROWS)}


def _ln_rows(y, g, b):
    mu = jnp.mean(y, axis=-1, keepdims=True)
    var = jnp.mean((y - mu) ** 2, axis=-1, keepdims=True)
    return (y - mu) / jnp.sqrt(var + 1e-5) * g + b


def _full(shape):
    return pl.BlockSpec(shape, lambda i: tuple(0 for _ in shape))


def _tc_encoder(x, W0, b0, Wout, bout, g, b, blk):
    n, din = x.shape

    def body(x_ref, W0_ref, b0_ref, Wout_ref, bout_ref, g_ref, b_ref, out_ref):
        h = jnp.maximum(jnp.dot(x_ref[...], W0_ref[...], precision=_PREC,
                                preferred_element_type=jnp.float32) + b0_ref[...], 0.0)
        y = jnp.dot(h, Wout_ref[...], precision=_PREC,
                    preferred_element_type=jnp.float32) + bout_ref[...]
        out_ref[...] = _ln_rows(y, g_ref[...], b_ref[...])

    return pl.pallas_call(
        body,
        grid=(n // blk,),
        in_specs=[pl.BlockSpec((blk, din), lambda i: (i, 0)),
                  _full((din, L)), _full((1, L)), _full((L, L)), _full((1, L)),
                  _full((1, L)), _full((1, L))],
        out_specs=pl.BlockSpec((blk, L), lambda i: (i, 0)),
        out_shape=jax.ShapeDtypeStruct((n, L), jnp.float32),
    )(x, W0, b0.reshape(1, L), Wout, bout.reshape(1, L), g.reshape(1, L), b.reshape(1, L))


def _tc_node_prod(vlat, Wd, Ws):
    def body(v_ref, wd_ref, ws_ref, ad_ref, as_ref):
        v = v_ref[...]
        ad_ref[...] = jnp.dot(v, wd_ref[...], precision=_PREC, preferred_element_type=jnp.float32)
        as_ref[...] = jnp.dot(v, ws_ref[...], precision=_PREC, preferred_element_type=jnp.float32)

    return pl.pallas_call(
        body,
        grid=(N // BN,),
        in_specs=[pl.BlockSpec((BN, L), lambda i: (i, 0)), _full((L, L)), _full((L, L))],
        out_specs=(pl.BlockSpec((BN, L), lambda i: (i, 0)), pl.BlockSpec((BN, L), lambda i: (i, 0))),
        out_shape=(jax.ShapeDtypeStruct((N, L), jnp.float32),
                   jax.ShapeDtypeStruct((N, L), jnp.float32)),
    )(vlat, Wd, Ws)


def _tc_edge_update(G, elat, We, b0, Wout, bout, attW, attb, g, b):
    nE = G.shape[0]

    def body(g_ref, e_ref, We_ref, b0_ref, Wout_ref, bout_ref, aw_ref, ab_ref,
             g_ref2, b_ref2, ne_ref, wne_ref, ex_ref):
        el = e_ref[...]
        hid = jnp.maximum(g_ref[...] + jnp.dot(el, We_ref[...], precision=_PREC,
                                               preferred_element_type=jnp.float32) + b0_ref[...], 0.0)
        eup = jnp.dot(hid, Wout_ref[...], precision=_PREC,
                      preferred_element_type=jnp.float32) + bout_ref[...]
        ne = el + _ln_rows(eup, g_ref2[...], b_ref2[...])
        s = jnp.maximum(jnp.sum(el * aw_ref[...], axis=-1, keepdims=True) + ab_ref[...], 0.0)
        ex = jnp.exp(s)
        ne_ref[...] = ne
        wne_ref[...] = ex * ne
        ex_ref[...] = ex.reshape(1, BE // 128, 128)

    return pl.pallas_call(
        body,
        grid=(nE // BE,),
        in_specs=[pl.BlockSpec((BE, L), lambda i: (i, 0)),
                  pl.BlockSpec((BE, L), lambda i: (i, 0)),
                  _full((L, L)), _full((1, L)), _full((L, L)), _full((1, L)),
                  _full((1, L)), _full((1, 1)), _full((1, L)), _full((1, L))],
        out_specs=(pl.BlockSpec((BE, L), lambda i: (i, 0)),
                   pl.BlockSpec((BE, L), lambda i: (i, 0)),
                   pl.BlockSpec((1, BE // 128, 128), lambda i: (i, 0, 0))),
        out_shape=(jax.ShapeDtypeStruct((nE, L), jnp.float32),
                   jax.ShapeDtypeStruct((nE, L), jnp.float32),
                   jax.ShapeDtypeStruct((nE // BE, BE // 128, 128), jnp.float32)),
    )(G, elat, We, b0.reshape(1, L), Wout, bout.reshape(1, L),
      attW.reshape(1, L), attb.reshape(1, 1), g.reshape(1, L), b.reshape(1, L))


def _tc_vertex_update(vlat, vps, dps, Wv1, Wv2, b0, Wout, bout, g, b, Wdn, Wsn):
    nv = len(vps)
    nd = len(dps)

    def body(*refs):
        v_ref = refs[0]
        vp_refs = refs[1:1 + nv]
        dp_refs = refs[1 + nv:1 + nv + nd]
        (Wv1_ref, Wv2_ref, b0_ref, Wout_ref, bout_ref,
         g_ref, b_ref, wdn_ref, wsn_ref) = refs[1 + nv + nd:1 + nv + nd + 9]
        vo_ref, ad_ref, as_ref = refs[-3:]
        v = v_ref[...]
        den = sum(jnp.sum(d[...], axis=-1, keepdims=True) for d in dp_refs)
        vals = sum(p[...] for p in vp_refs)
        agg = vals / (den + 1e-16)
        hv = jnp.maximum(jnp.dot(v, Wv1_ref[...], precision=_PREC, preferred_element_type=jnp.float32)
                         + jnp.dot(agg, Wv2_ref[...], precision=_PREC, preferred_element_type=jnp.float32)
                         + b0_ref[...], 0.0)
        vup = jnp.dot(hv, Wout_ref[...], precision=_PREC,
                      preferred_element_type=jnp.float32) + bout_ref[...]
        vnew = v + _ln_rows(vup, g_ref[...], b_ref[...])
        vo_ref[...] = vnew
        ad_ref[...] = jnp.dot(vnew, wdn_ref[...], precision=_PREC, preferred_element_type=jnp.float32)
        as_ref[...] = jnp.dot(vnew, wsn_ref[...], precision=_PREC, preferred_element_type=jnp.float32)

    vspec = pl.BlockSpec((BN, L), lambda i: (i, 0))
    dspec = pl.BlockSpec((BN, NW), lambda i: (i, 0))
    return pl.pallas_call(
        body,
        grid=(N // BN,),
        in_specs=[vspec] + [vspec] * nv + [dspec] * nd +
                 [_full((L, L)), _full((L, L)), _full((1, L)), _full((L, L)),
                  _full((1, L)), _full((1, L)), _full((1, L)), _full((L, L)), _full((L, L))],
        out_specs=(vspec, vspec, vspec),
        out_shape=(jax.ShapeDtypeStruct((N, L), jnp.float32),
                   jax.ShapeDtypeStruct((N, L), jnp.float32),
                   jax.ShapeDtypeStruct((N, L), jnp.float32)),
    )(vlat, *vps, *dps, Wv1, Wv2, b0.reshape(1, L), Wout, bout.reshape(1, L),
      g.reshape(1, L), b.reshape(1, L), Wdn, Wsn)


def _tc_decoder(vlat, W0, b0, Wout, bout):
    def body(v_ref, W0_ref, b0_ref, Wout_ref, bout_ref, out_ref):
        h = jnp.maximum(jnp.dot(v_ref[...], W0_ref[...], precision=_PREC,
                                preferred_element_type=jnp.float32) + b0_ref[...], 0.0)
        out_ref[...] = jnp.dot(h, Wout_ref[...], precision=_PREC,
                               preferred_element_type=jnp.float32) + bout_ref[...]

    Wout_p = jnp.zeros((L, 128), jnp.float32).at[:, :OUT_DIM].set(Wout)
    bout_p = jnp.zeros((1, 128), jnp.float32).at[0, :OUT_DIM].set(bout)
    out = pl.pallas_call(
        body,
        grid=(N // BN,),
        in_specs=[pl.BlockSpec((BN, L), lambda i: (i, 0)),
                  _full((L, L)), _full((1, L)), _full((L, 128)), _full((1, 128))],
        out_specs=pl.BlockSpec((BN, 128), lambda i: (i, 0)),
        out_shape=jax.ShapeDtypeStruct((N, 128), jnp.float32),
    )(vlat, W0, b0.reshape(1, L), Wout_p, bout_p)
    return out[:, :OUT_DIM]


def _fold_bn(p):
    s = p['bn_g'] / jnp.sqrt(p['bn_v'] + 1e-5)
    t = p['bn_b'] - p['bn_m'] * s
    return s[:, None] * p['W0'], t @ p['W0'] + p['b0']


def kernel(x, edge_attr, edge_index, params):
    src_r = edge_index[0].reshape(ROWS, 128)
    dst_r = edge_index[1].reshape(ROWS, 128)
    src_c = [src_r[CHUNK_OFF[c]:CHUNK_OFF[c] + CHUNK_ROWS[c]] for c in range(NCHUNK)]
    dst_c = [dst_r[CHUNK_OFF[c]:CHUNK_OFF[c] + CHUNK_ROWS[c]] for c in range(NCHUNK)]

    W0v, b0v = _fold_bn(params['venc'])
    W0e, b0e = _fold_bn(params['eenc'])
    lng, lnb = params['ln_g'], params['ln_b']

    vlat = _tc_encoder(x, W0v, b0v, params['venc']['Wout'], params['venc']['bout'], lng, lnb, BN)
    elat_c = [
        _tc_encoder(edge_attr[CHUNK_OFF[c] * 128:(CHUNK_OFF[c] + CHUNK_ROWS[c]) * 128],
                    W0e, b0e, params['eenc']['Wout'], params['eenc']['bout'], lng, lnb, BE)
        for c in range(NCHUNK)
    ]

    pp0 = params['proc0']
    Ad, As = _tc_node_prod(vlat, pp0['edge']['W0'][0:L], pp0['edge']['W0'][L:2 * L])

    for i in range(MP):
        pp = params['proc%d' % i]
        ppn = params['proc%d' % ((i + 1) % MP)]
        ne_c, vp_list, dp_list = [], [], []
        G_c = [None] * NCHUNK
        for c in range(NCHUNK):
            G_c[c] = _sc_gather_c[CHUNK_ROWS[c]](Ad, As, dst_c[c], src_c[c])
        for c in range(NCHUNK):
            ne, wne, exr = _tc_edge_update(
                G_c[c], elat_c[c], pp['edge']['W0'][2 * L:3 * L], pp['edge']['b0'],
                pp['edge']['Wout'], pp['edge']['bout'],
                pp['att_W'][:, 0], pp['att_b'], pp['ln_g'], pp['ln_b'])
            ne_c.append(ne)
            vparts, dparts = _sc_scatter_c[CHUNK_ROWS[c]](wne, exr.reshape(CHUNK_ROWS[c], 128), dst_c[c])
            vp_list.extend([vparts[0, :N], vparts[1, :N]])
            dp_list.append(dparts.T)
        vlat, Ad, As = _tc_vertex_update(
            vlat, vp_list, dp_list,
            pp['vertex']['W0'][0:L], pp['vertex']['W0'][L:2 * L],
            pp['vertex']['b0'], pp['vertex']['Wout'], pp['vertex']['bout'],
            pp['ln_g'], pp['ln_b'],
            ppn['edge']['W0'][0:L], ppn['edge']['W0'][L:2 * L])
        elat_c = ne_c

    d = params['dec']
    return _tc_decoder(vlat, d['W0'], d['b0'], d['Wout'], d['bout'])
